# baseline (speedup 1.0000x reference)
import functools

import jax
import jax.numpy as jnp
from jax import lax
from jax.experimental import pallas as pl
from jax.experimental.pallas import tpu as pltpu
from jax.experimental.pallas import tpu_sc as plsc

B, S, D = 1, 2048, 768
H = 4 * D
E = 8
T = B * S
K = 2
NPAIR = K * T
BLK = 128
NBLK = NPAIR // BLK + E
P = NBLK * BLK


def _gating_meta_body(x_ref, wg_ref, g_ref, pos_ref, src_ref, be_ref, o_ref):
    x = x_ref[...]
    logits = jnp.dot(x, wg_ref[...], preferred_element_type=jnp.float32)
    iota8 = lax.broadcasted_iota(jnp.int32, (T, E), 1)
    m1 = jnp.max(logits, axis=1, keepdims=True)
    e1 = jnp.min(jnp.where(logits == m1, iota8, E), axis=1, keepdims=True)
    masked = jnp.where(iota8 == e1, -jnp.inf, logits)
    m2 = jnp.max(masked, axis=1, keepdims=True)
    e2 = jnp.min(jnp.where(masked == m2, iota8, E), axis=1, keepdims=True)
    g1 = 1.0 / (1.0 + jnp.exp(m2 - m1))
    g_ref[...] = jnp.concatenate([g1, 1.0 - g1], axis=1)

    one1 = (iota8 == e1).astype(jnp.float32)
    one2 = (iota8 == e2).astype(jnp.float32)
    o_ref[0:T, :] = one1
    o_ref[T:NPAIR, :] = one2

    cnt = jnp.sum(one1, axis=0, keepdims=True) + jnp.sum(one2, axis=0, keepdims=True)
    cnt_i = cnt.astype(jnp.int32)
    padded = (((cnt_i + (BLK - 1)) // BLK) * BLK).astype(jnp.float32)
    ir = lax.broadcasted_iota(jnp.int32, (E, E), 0)
    ic = lax.broadcasted_iota(jnp.int32, (E, E), 1)
    ut8 = (ir < ic).astype(jnp.float32)
    off = jnp.dot(padded, ut8, preferred_element_type=jnp.float32)

    lr = lax.broadcasted_iota(jnp.int32, (BLK, BLK), 0)
    lc = lax.broadcasted_iota(jnp.int32, (BLK, BLK), 1)
    lt = (lc < lr).astype(jnp.float32)

    def rank_step(i, rc):
        ob = o_ref[pl.ds(i * BLK, BLK), :]
        cb = jnp.dot(lt, ob, preferred_element_type=jnp.float32) + rc
        posb = jnp.sum((cb + off) * ob, axis=1, keepdims=True)
        pos_ref[pl.ds(i * BLK, BLK), :] = posb.astype(jnp.int32)
        return rc + jnp.sum(ob, axis=0, keepdims=True)

    lax.fori_loop(0, NPAIR // BLK, rank_step, jnp.zeros((1, E), jnp.float32))

    tv = lax.broadcasted_iota(jnp.int32, (1, NPAIR), 1) % T
    tlo = (tv % 8).astype(jnp.float32)
    thi = (tv // 8).astype(jnp.float32)
    pos_col = pos_ref[...]

    def src_step(b, carry):
        crow = b * BLK + lax.broadcasted_iota(jnp.int32, (1, BLK), 1)
        mb = (pos_col == crow).astype(jnp.float32)
        srcb = (jnp.dot(tlo, mb, preferred_element_type=jnp.float32)
                + 8.0 * jnp.dot(thi, mb, preferred_element_type=jnp.float32))
        hit = jnp.sum(mb, axis=0, keepdims=True)
        srcb = srcb + (1.0 - hit) * (crow % T).astype(jnp.float32)
        srcb = jnp.clip(srcb, 0.0, float(T - 1))
        src_ref[pl.ds(b, 1), :] = srcb.astype(jnp.int32)
        return carry

    lax.fori_loop(0, NBLK, src_step, jnp.int32(0))

    bstart = (BLK * lax.broadcasted_iota(jnp.int32, (1, NBLK), 1)).astype(jnp.float32)
    acc = jnp.zeros((1, NBLK), jnp.float32)
    any_ind = jnp.zeros((1, NBLK), jnp.float32)
    for e in range(E):
        lo = off[:, e:e + 1]
        hi = lo + padded[:, e:e + 1]
        ind = ((bstart >= lo) & (bstart < hi)).astype(jnp.float32)
        acc = acc + e * ind
        any_ind = any_ind + ind
    be = acc + (E - 1) * (1.0 - any_ind)
    be_ref[...] = be.astype(jnp.int32)


def _gating_meta(x_flat, Wg):
    g, pos, src, be = pl.pallas_call(
        _gating_meta_body,
        out_shape=(
            jax.ShapeDtypeStruct((T, K), jnp.float32),
            jax.ShapeDtypeStruct((NPAIR, 1), jnp.int32),
            jax.ShapeDtypeStruct((NBLK, BLK), jnp.int32),
            jax.ShapeDtypeStruct((1, NBLK), jnp.int32),
        ),
        scratch_shapes=[pltpu.VMEM((NPAIR, E), jnp.float32)],
    )(x_flat, Wg)
    return g, pos.reshape(NPAIR), src.reshape(P), be.reshape(NBLK)


def _sc_scatter_x():
    nc, ns = 2, 16
    b_per_w = NPAIR // (nc * ns)

    mesh = plsc.VectorSubcoreMesh(core_axis_name="c", subcore_axis_name="s")

    @functools.partial(
        pl.kernel, mesh=mesh,
        out_type=jax.ShapeDtypeStruct((P, D), jnp.float32),
        scratch_types=[
            pltpu.VMEM((b_per_w,), jnp.int32),
            pltpu.VMEM((b_per_w, D), jnp.float32),
            pltpu.SemaphoreType.DMA,
        ],
    )
    def scatter(x_hbm, pos_hbm, out_hbm, idx_v, rows_v, sem):
        wid = lax.axis_index("s") * nc + lax.axis_index("c")
        base = wid * b_per_w
        trow = lax.rem(base, T)
        pltpu.sync_copy(pos_hbm.at[pl.ds(base, b_per_w)], idx_v)
        pltpu.sync_copy(x_hbm.at[pl.ds(trow, b_per_w)], rows_v)
        pltpu.async_copy(rows_v, out_hbm.at[idx_v], sem).wait()

    return scatter


def _grouped_body(be_ref, x_ref, w1_ref, w2_ref, y_ref):
    h = jnp.dot(x_ref[...], w1_ref[0], preferred_element_type=jnp.float32)
    h = h / (1.0 + jnp.exp(-h))
    y_ref[...] = jnp.dot(h, w2_ref[0], preferred_element_type=jnp.float32)


def _grouped_matmul(x_sorted, We1, We2, be):
    grid_spec = pltpu.PrefetchScalarGridSpec(
        num_scalar_prefetch=1,
        grid=(NBLK,),
        in_specs=[
            pl.BlockSpec((BLK, D), lambda i, be_ref: (i, 0)),
            pl.BlockSpec((1, D, H), lambda i, be_ref: (be_ref[i], 0, 0)),
            pl.BlockSpec((1, H, D), lambda i, be_ref: (be_ref[i], 0, 0)),
        ],
        out_specs=pl.BlockSpec((BLK, D), lambda i, be_ref: (i, 0)),
    )
    return pl.pallas_call(
        _grouped_body,
        grid_spec=grid_spec,
        out_shape=jax.ShapeDtypeStruct((P, D), jnp.float32),
    )(be, x_sorted, We1, We2)


def _make_sc_gather(n_rows, b_total):
    nc, ns = 2, 16
    nw = nc * ns
    b_per_w = b_total // nw
    n_chunks = -(-b_per_w // 128)
    c = b_per_w // n_chunks
    assert c * n_chunks == b_per_w and c % 8 == 0
    mesh = plsc.VectorSubcoreMesh(core_axis_name="c", subcore_axis_name="s")

    @functools.partial(
        pl.kernel, mesh=mesh,
        out_type=jax.ShapeDtypeStruct((b_total, D), jnp.float32),
        scratch_types=[
            [pltpu.VMEM((c,), jnp.int32) for _ in range(n_chunks)],
            [pltpu.VMEM((c, D), jnp.float32) for _ in range(n_chunks)],
            [pltpu.SemaphoreType.DMA for _ in range(n_chunks)],
            [pltpu.SemaphoreType.DMA for _ in range(n_chunks)],
        ],
    )
    def gather(table_hbm, idx_hbm, out_hbm, idx_vs, rows_vs, gsems, osems):
        wid = lax.axis_index("s") * nc + lax.axis_index("c")
        base = wid * b_per_w
        gathers = []
        for j in range(n_chunks):
            pltpu.sync_copy(idx_hbm.at[pl.ds(base + j * c, c)], idx_vs[j])
            gathers.append(
                pltpu.async_copy(table_hbm.at[idx_vs[j]], rows_vs[j], gsems[j]))
        outs = []
        for j in range(n_chunks):
            gathers[j].wait()
            outs.append(
                pltpu.async_copy(rows_vs[j], out_hbm.at[pl.ds(base + j * c, c)],
                                 osems[j]))
        for cp in outs:
            cp.wait()

    return gather


def _sc_gather(table, idx, b_total):
    return _make_sc_gather(table.shape[0], b_total)(table, idx)


def _shared_body(x_ref, ws1_ref, ws2_ref, out_ref):
    h = jnp.dot(x_ref[...], ws1_ref[...], preferred_element_type=jnp.float32)
    h = h / (1.0 + jnp.exp(-h))
    out_ref[...] = jnp.dot(h, ws2_ref[...], preferred_element_type=jnp.float32)


def _shared_expert_half(x_flat, Ws1h, Ws2h):
    BT = 512
    nt = T // BT
    hh = Ws1h.shape[1]
    return pl.pallas_call(
        _shared_body,
        grid=(nt,),
        in_specs=[
            pl.BlockSpec((BT, D), lambda i: (i, 0)),
            pl.BlockSpec((D, hh), lambda i: (0, 0)),
            pl.BlockSpec((hh, D), lambda i: (0, 0)),
        ],
        out_specs=pl.BlockSpec((BT, D), lambda i: (i, 0)),
        out_shape=jax.ShapeDtypeStruct((T, D), jnp.float32),
    )(x_flat, Ws1h, Ws2h)


def _combine_body(s0_ref, s1_ref, z1_ref, z2_ref, g_ref, out_ref):
    out_ref[...] = (s0_ref[...] + s1_ref[...] + g_ref[:, 0:1] * z1_ref[...]
                    + g_ref[:, 1:2] * z2_ref[...])


def _combine(sh0, sh1, z, g):
    BT = 1024
    nt = T // BT
    return pl.pallas_call(
        _combine_body,
        grid=(nt,),
        in_specs=[
            pl.BlockSpec((BT, D), lambda i: (i, 0)),
            pl.BlockSpec((BT, D), lambda i: (i, 0)),
            pl.BlockSpec((BT, D), lambda i: (i, 0)),
            pl.BlockSpec((BT, D), lambda i: (i + nt, 0)),
            pl.BlockSpec((BT, K), lambda i: (i, 0)),
        ],
        out_specs=pl.BlockSpec((BT, D), lambda i: (i, 0)),
        out_shape=jax.ShapeDtypeStruct((T, D), jnp.float32),
    )(sh0, sh1, z, z, g)


def kernel(x, Wg, Ws1, Ws2, We1, We2):
    x_flat = x.reshape(T, D)
    bf = jnp.bfloat16
    HH = H // 2
    g, pos, src, be = _gating_meta(x_flat, Wg)
    x_sorted = _sc_gather(x_flat, src, P)
    sh0 = _shared_expert_half(x_flat, Ws1[:, :HH].astype(bf), Ws2[:HH].astype(bf))
    y_sorted = _grouped_matmul(x_sorted, We1.astype(bf), We2.astype(bf), be)
    z = _sc_gather(y_sorted, pos, NPAIR)
    sh1 = _shared_expert_half(x_flat, Ws1[:, HH:].astype(bf), Ws2[HH:].astype(bf))
    out = _combine(sh0, sh1, z, g)
    return out.reshape(B, S, D)


# speedup vs baseline: 1.0113x; 1.0113x over previous
import functools

import jax
import jax.numpy as jnp
from jax import lax
from jax.experimental import pallas as pl
from jax.experimental.pallas import tpu as pltpu
from jax.experimental.pallas import tpu_sc as plsc

B, S, D = 1, 2048, 768
H = 4 * D
E = 8
T = B * S
K = 2
NPAIR = K * T
BLK = 128
NBLK = NPAIR // BLK + E
P = NBLK * BLK


def _gating_meta_body(x_ref, wg_ref, g_ref, pos_ref, src_ref, be_ref, o_ref):
    x = x_ref[...]
    logits = jnp.dot(x, wg_ref[...], preferred_element_type=jnp.float32)
    iota8 = lax.broadcasted_iota(jnp.int32, (T, E), 1)
    m1 = jnp.max(logits, axis=1, keepdims=True)
    e1 = jnp.min(jnp.where(logits == m1, iota8, E), axis=1, keepdims=True)
    masked = jnp.where(iota8 == e1, -jnp.inf, logits)
    m2 = jnp.max(masked, axis=1, keepdims=True)
    e2 = jnp.min(jnp.where(masked == m2, iota8, E), axis=1, keepdims=True)
    g1 = 1.0 / (1.0 + jnp.exp(m2 - m1))
    g_ref[...] = jnp.concatenate([g1, 1.0 - g1], axis=1)

    one1 = (iota8 == e1).astype(jnp.float32)
    one2 = (iota8 == e2).astype(jnp.float32)
    o_ref[0:T, :] = one1
    o_ref[T:NPAIR, :] = one2

    cnt = jnp.sum(one1, axis=0, keepdims=True) + jnp.sum(one2, axis=0, keepdims=True)
    cnt_i = cnt.astype(jnp.int32)
    padded = (((cnt_i + (BLK - 1)) // BLK) * BLK).astype(jnp.float32)
    ir = lax.broadcasted_iota(jnp.int32, (E, E), 0)
    ic = lax.broadcasted_iota(jnp.int32, (E, E), 1)
    ut8 = (ir < ic).astype(jnp.float32)
    off = jnp.dot(padded, ut8, preferred_element_type=jnp.float32)

    lr = lax.broadcasted_iota(jnp.int32, (BLK, BLK), 0)
    lc = lax.broadcasted_iota(jnp.int32, (BLK, BLK), 1)
    lt = (lc < lr).astype(jnp.float32)

    def rank_step(i, rc):
        ob = o_ref[pl.ds(i * BLK, BLK), :]
        cb = jnp.dot(lt, ob, preferred_element_type=jnp.float32) + rc
        posb = jnp.sum((cb + off) * ob, axis=1, keepdims=True)
        pos_ref[pl.ds(i * BLK, BLK), :] = posb.astype(jnp.int32)
        return rc + jnp.sum(ob, axis=0, keepdims=True)

    lax.fori_loop(0, NPAIR // BLK, rank_step, jnp.zeros((1, E), jnp.float32))

    tvals = (lax.broadcasted_iota(jnp.int32, (1, NPAIR), 1) % T).astype(jnp.float32)
    pos_col = pos_ref[...]

    def src_step(b, carry):
        crow = b * BLK + lax.broadcasted_iota(jnp.int32, (1, BLK), 1)
        mb = (pos_col == crow).astype(jnp.float32)
        srcb = jnp.dot(tvals, mb, preferred_element_type=jnp.float32,
                       precision=jax.lax.Precision.HIGHEST)
        hit = jnp.sum(mb, axis=0, keepdims=True)
        srcb = srcb + (1.0 - hit) * (crow % T).astype(jnp.float32)
        srcb = jnp.clip(srcb, 0.0, float(T - 1))
        src_ref[pl.ds(b, 1), :] = srcb.astype(jnp.int32)
        return carry

    lax.fori_loop(0, NBLK, src_step, jnp.int32(0))

    bstart = (BLK * lax.broadcasted_iota(jnp.int32, (1, NBLK), 1)).astype(jnp.float32)
    acc = jnp.zeros((1, NBLK), jnp.float32)
    any_ind = jnp.zeros((1, NBLK), jnp.float32)
    for e in range(E):
        lo = off[:, e:e + 1]
        hi = lo + padded[:, e:e + 1]
        ind = ((bstart >= lo) & (bstart < hi)).astype(jnp.float32)
        acc = acc + e * ind
        any_ind = any_ind + ind
    be = acc + (E - 1) * (1.0 - any_ind)
    be_ref[...] = be.astype(jnp.int32)


def _gating_meta(x_flat, Wg):
    g, pos, src, be = pl.pallas_call(
        _gating_meta_body,
        out_shape=(
            jax.ShapeDtypeStruct((T, K), jnp.float32),
            jax.ShapeDtypeStruct((NPAIR, 1), jnp.int32),
            jax.ShapeDtypeStruct((NBLK, BLK), jnp.int32),
            jax.ShapeDtypeStruct((1, NBLK), jnp.int32),
        ),
        scratch_shapes=[pltpu.VMEM((NPAIR, E), jnp.float32)],
    )(x_flat, Wg)
    return g, pos.reshape(NPAIR), src.reshape(P), be.reshape(NBLK)


def _sc_scatter_x():
    nc, ns = 2, 16
    b_per_w = NPAIR // (nc * ns)

    mesh = plsc.VectorSubcoreMesh(core_axis_name="c", subcore_axis_name="s")

    @functools.partial(
        pl.kernel, mesh=mesh,
        out_type=jax.ShapeDtypeStruct((P, D), jnp.float32),
        scratch_types=[
            pltpu.VMEM((b_per_w,), jnp.int32),
            pltpu.VMEM((b_per_w, D), jnp.float32),
            pltpu.SemaphoreType.DMA,
        ],
    )
    def scatter(x_hbm, pos_hbm, out_hbm, idx_v, rows_v, sem):
        wid = lax.axis_index("s") * nc + lax.axis_index("c")
        base = wid * b_per_w
        trow = lax.rem(base, T)
        pltpu.sync_copy(pos_hbm.at[pl.ds(base, b_per_w)], idx_v)
        pltpu.sync_copy(x_hbm.at[pl.ds(trow, b_per_w)], rows_v)
        pltpu.async_copy(rows_v, out_hbm.at[idx_v], sem).wait()

    return scatter


def _grouped_body(be_ref, x_ref, w1_ref, w2_ref, y_ref):
    h = jnp.dot(x_ref[...], w1_ref[0], preferred_element_type=jnp.float32)
    h = h / (1.0 + jnp.exp(-h))
    y_ref[...] = jnp.dot(h, w2_ref[0], preferred_element_type=jnp.float32)


def _grouped_matmul(x_sorted, We1, We2, be):
    grid_spec = pltpu.PrefetchScalarGridSpec(
        num_scalar_prefetch=1,
        grid=(NBLK,),
        in_specs=[
            pl.BlockSpec((BLK, D), lambda i, be_ref: (i, 0)),
            pl.BlockSpec((1, D, H), lambda i, be_ref: (be_ref[i], 0, 0)),
            pl.BlockSpec((1, H, D), lambda i, be_ref: (be_ref[i], 0, 0)),
        ],
        out_specs=pl.BlockSpec((BLK, D), lambda i, be_ref: (i, 0)),
    )
    return pl.pallas_call(
        _grouped_body,
        grid_spec=grid_spec,
        out_shape=jax.ShapeDtypeStruct((P, D), jnp.float32),
    )(be, x_sorted, We1, We2)


def _make_sc_gather(n_rows, b_total):
    nc, ns = 2, 16
    nw = nc * ns
    b_per_w = b_total // nw
    n_chunks = -(-b_per_w // 128)
    c = b_per_w // n_chunks
    assert c * n_chunks == b_per_w and c % 8 == 0
    mesh = plsc.VectorSubcoreMesh(core_axis_name="c", subcore_axis_name="s")

    @functools.partial(
        pl.kernel, mesh=mesh,
        out_type=jax.ShapeDtypeStruct((b_total, D), jnp.float32),
        scratch_types=[
            [pltpu.VMEM((c,), jnp.int32) for _ in range(n_chunks)],
            [pltpu.VMEM((c, D), jnp.float32) for _ in range(n_chunks)],
            [pltpu.SemaphoreType.DMA for _ in range(n_chunks)],
            [pltpu.SemaphoreType.DMA for _ in range(n_chunks)],
        ],
    )
    def gather(table_hbm, idx_hbm, out_hbm, idx_vs, rows_vs, gsems, osems):
        wid = lax.axis_index("s") * nc + lax.axis_index("c")
        base = wid * b_per_w
        gathers = []
        for j in range(n_chunks):
            pltpu.sync_copy(idx_hbm.at[pl.ds(base + j * c, c)], idx_vs[j])
            gathers.append(
                pltpu.async_copy(table_hbm.at[idx_vs[j]], rows_vs[j], gsems[j]))
        outs = []
        for j in range(n_chunks):
            gathers[j].wait()
            outs.append(
                pltpu.async_copy(rows_vs[j], out_hbm.at[pl.ds(base + j * c, c)],
                                 osems[j]))
        for cp in outs:
            cp.wait()

    return gather


def _sc_gather(table, idx, b_total):
    return _make_sc_gather(table.shape[0], b_total)(table, idx)


def _shared_body(x_ref, ws1_ref, ws2_ref, out_ref):
    h = jnp.dot(x_ref[...], ws1_ref[...], preferred_element_type=jnp.float32)
    h = h / (1.0 + jnp.exp(-h))
    out_ref[...] = jnp.dot(h, ws2_ref[...], preferred_element_type=jnp.float32)


def _shared_expert_half(x_flat, Ws1h, Ws2h):
    BT = 512
    nt = T // BT
    hh = Ws1h.shape[1]
    return pl.pallas_call(
        _shared_body,
        grid=(nt,),
        in_specs=[
            pl.BlockSpec((BT, D), lambda i: (i, 0)),
            pl.BlockSpec((D, hh), lambda i: (0, 0)),
            pl.BlockSpec((hh, D), lambda i: (0, 0)),
        ],
        out_specs=pl.BlockSpec((BT, D), lambda i: (i, 0)),
        out_shape=jax.ShapeDtypeStruct((T, D), jnp.float32),
    )(x_flat, Ws1h, Ws2h)


def _shared_combine_body(x_ref, ws1_ref, ws2_ref, z1_ref, z2_ref, g_ref, out_ref):
    h = jnp.dot(x_ref[...], ws1_ref[...], preferred_element_type=jnp.float32)
    h = h / (1.0 + jnp.exp(-h))
    sh = jnp.dot(h, ws2_ref[...], preferred_element_type=jnp.float32)
    out_ref[...] = (sh + g_ref[:, 0:1] * z1_ref[...] + g_ref[:, 1:2] * z2_ref[...])


def _shared_combine(x_flat, Ws1, Ws2, z, g):
    BT = 256
    nt = T // BT
    return pl.pallas_call(
        _shared_combine_body,
        grid=(nt,),
        in_specs=[
            pl.BlockSpec((BT, D), lambda i: (i, 0)),
            pl.BlockSpec((D, H), lambda i: (0, 0)),
            pl.BlockSpec((H, D), lambda i: (0, 0)),
            pl.BlockSpec((BT, D), lambda i: (i, 0)),
            pl.BlockSpec((BT, D), lambda i: (i + nt, 0)),
            pl.BlockSpec((BT, K), lambda i: (i, 0)),
        ],
        out_specs=pl.BlockSpec((BT, D), lambda i: (i, 0)),
        out_shape=jax.ShapeDtypeStruct((T, D), jnp.float32),
    )(x_flat, Ws1, Ws2, z, z, g)


def kernel(x, Wg, Ws1, Ws2, We1, We2):
    x_flat = x.reshape(T, D)
    bf = jnp.bfloat16
    HH = H // 2
    g, pos, src, be = _gating_meta(x_flat, Wg)
    x_sorted = _sc_gather(x_flat, src, P)
    y_sorted = _grouped_matmul(x_sorted, We1.astype(bf), We2.astype(bf), be)
    z = _sc_gather(y_sorted, pos, NPAIR)
    out = _shared_combine(x_flat, Ws1.astype(bf), Ws2.astype(bf), z, g)
    return out.reshape(B, S, D)


# speedup vs baseline: 1.2165x; 1.2029x over previous
import functools

import jax
import jax.numpy as jnp
from jax import lax
from jax.experimental import pallas as pl
from jax.experimental.pallas import tpu as pltpu
from jax.experimental.pallas import tpu_sc as plsc

B, S, D = 1, 2048, 768
H = 4 * D
E = 8
T = B * S
K = 2
NPAIR = K * T
BLK = 128
NBLK = NPAIR // BLK + E
P = NBLK * BLK


def _gating_meta_body(x_ref, wg_ref, g_ref, pos_ref, src_ref, be_ref, o_ref):
    x = x_ref[...]
    logits = jnp.dot(x, wg_ref[...], preferred_element_type=jnp.float32)
    iota8 = lax.broadcasted_iota(jnp.int32, (T, E), 1)
    m1 = jnp.max(logits, axis=1, keepdims=True)
    e1 = jnp.min(jnp.where(logits == m1, iota8, E), axis=1, keepdims=True)
    masked = jnp.where(iota8 == e1, -jnp.inf, logits)
    m2 = jnp.max(masked, axis=1, keepdims=True)
    e2 = jnp.min(jnp.where(masked == m2, iota8, E), axis=1, keepdims=True)
    g1 = 1.0 / (1.0 + jnp.exp(m2 - m1))
    g_ref[...] = jnp.concatenate([g1, 1.0 - g1], axis=1)

    one1 = (iota8 == e1).astype(jnp.float32)
    one2 = (iota8 == e2).astype(jnp.float32)
    o_ref[0:T, :] = one1
    o_ref[T:NPAIR, :] = one2

    cnt = jnp.sum(one1, axis=0, keepdims=True) + jnp.sum(one2, axis=0, keepdims=True)
    cnt_i = cnt.astype(jnp.int32)
    padded = (((cnt_i + (BLK - 1)) // BLK) * BLK).astype(jnp.float32)
    ir = lax.broadcasted_iota(jnp.int32, (E, E), 0)
    ic = lax.broadcasted_iota(jnp.int32, (E, E), 1)
    ut8 = (ir < ic).astype(jnp.float32)
    off = jnp.dot(padded, ut8, preferred_element_type=jnp.float32)

    lr = lax.broadcasted_iota(jnp.int32, (BLK, BLK), 0)
    lc = lax.broadcasted_iota(jnp.int32, (BLK, BLK), 1)
    lt = (lc < lr).astype(jnp.float32)

    def rank_step(i, rc):
        ob = o_ref[pl.ds(i * BLK, BLK), :]
        cb = jnp.dot(lt, ob, preferred_element_type=jnp.float32) + rc
        posb = jnp.sum((cb + off) * ob, axis=1, keepdims=True)
        pos_ref[pl.ds(i * BLK, BLK), :] = posb.astype(jnp.int32)
        return rc + jnp.sum(ob, axis=0, keepdims=True)

    lax.fori_loop(0, NPAIR // BLK, rank_step, jnp.zeros((1, E), jnp.float32))

    tvals = (lax.broadcasted_iota(jnp.int32, (1, NPAIR), 1) % T).astype(jnp.float32)
    pos_col = pos_ref[...]

    def src_step(b, carry):
        crow = b * BLK + lax.broadcasted_iota(jnp.int32, (1, BLK), 1)
        mb = (pos_col == crow).astype(jnp.float32)
        srcb = jnp.dot(tvals, mb, preferred_element_type=jnp.float32,
                       precision=jax.lax.Precision.HIGHEST)
        hit = jnp.sum(mb, axis=0, keepdims=True)
        srcb = srcb + (1.0 - hit) * (crow % T).astype(jnp.float32)
        srcb = jnp.clip(srcb, 0.0, float(T - 1))
        src_ref[pl.ds(b, 1), :] = srcb.astype(jnp.int32)
        return carry

    lax.fori_loop(0, NBLK, src_step, jnp.int32(0))

    bstart = (BLK * lax.broadcasted_iota(jnp.int32, (1, NBLK), 1)).astype(jnp.float32)
    acc = jnp.zeros((1, NBLK), jnp.float32)
    any_ind = jnp.zeros((1, NBLK), jnp.float32)
    for e in range(E):
        lo = off[:, e:e + 1]
        hi = lo + padded[:, e:e + 1]
        ind = ((bstart >= lo) & (bstart < hi)).astype(jnp.float32)
        acc = acc + e * ind
        any_ind = any_ind + ind
    be = acc + (E - 1) * (1.0 - any_ind)
    be_ref[...] = be.astype(jnp.int32)


def _gating_meta(x_flat, Wg):
    g, pos, src, be = pl.pallas_call(
        _gating_meta_body,
        out_shape=(
            jax.ShapeDtypeStruct((T, K), jnp.float32),
            jax.ShapeDtypeStruct((NPAIR, 1), jnp.int32),
            jax.ShapeDtypeStruct((NBLK, BLK), jnp.int32),
            jax.ShapeDtypeStruct((1, NBLK), jnp.int32),
        ),
        scratch_shapes=[pltpu.VMEM((NPAIR, E), jnp.float32)],
    )(x_flat, Wg)
    return g, pos.reshape(NPAIR), src.reshape(P), be.reshape(NBLK)


def _sc_scatter_x():
    nc, ns = 2, 16
    b_per_w = NPAIR // (nc * ns)

    mesh = plsc.VectorSubcoreMesh(core_axis_name="c", subcore_axis_name="s")

    @functools.partial(
        pl.kernel, mesh=mesh,
        out_type=jax.ShapeDtypeStruct((P, D), jnp.float32),
        scratch_types=[
            pltpu.VMEM((b_per_w,), jnp.int32),
            pltpu.VMEM((b_per_w, D), jnp.float32),
            pltpu.SemaphoreType.DMA,
        ],
    )
    def scatter(x_hbm, pos_hbm, out_hbm, idx_v, rows_v, sem):
        wid = lax.axis_index("s") * nc + lax.axis_index("c")
        base = wid * b_per_w
        trow = lax.rem(base, T)
        pltpu.sync_copy(pos_hbm.at[pl.ds(base, b_per_w)], idx_v)
        pltpu.sync_copy(x_hbm.at[pl.ds(trow, b_per_w)], rows_v)
        pltpu.async_copy(rows_v, out_hbm.at[idx_v], sem).wait()

    return scatter


def _grouped_body(be_ref, x_ref, w1_ref, w2_ref, y_ref):
    h = jnp.dot(x_ref[...], w1_ref[0], preferred_element_type=jnp.float32)
    h = h / (1.0 + jnp.exp(-h))
    y_ref[...] = jnp.dot(h, w2_ref[0], preferred_element_type=jnp.float32)


def _grouped_matmul(x_sorted, We1, We2, be):
    grid_spec = pltpu.PrefetchScalarGridSpec(
        num_scalar_prefetch=1,
        grid=(NBLK,),
        in_specs=[
            pl.BlockSpec((BLK, D), lambda i, be_ref: (i, 0)),
            pl.BlockSpec((1, D, H), lambda i, be_ref: (be_ref[i], 0, 0)),
            pl.BlockSpec((1, H, D), lambda i, be_ref: (be_ref[i], 0, 0)),
        ],
        out_specs=pl.BlockSpec((BLK, D), lambda i, be_ref: (i, 0)),
    )
    return pl.pallas_call(
        _grouped_body,
        grid_spec=grid_spec,
        out_shape=jax.ShapeDtypeStruct((P, D), jnp.float32),
    )(be, x_sorted, We1, We2)


def _make_sc_gather(n_rows, b_total):
    nc, ns = 2, 16
    nw = nc * ns
    b_per_w = b_total // nw
    n_chunks = -(-b_per_w // 128)
    c = b_per_w // n_chunks
    assert c * n_chunks == b_per_w and c % 8 == 0
    mesh = plsc.VectorSubcoreMesh(core_axis_name="c", subcore_axis_name="s")

    @functools.partial(
        pl.kernel, mesh=mesh,
        out_type=jax.ShapeDtypeStruct((b_total, D), jnp.float32),
        scratch_types=[
            [pltpu.VMEM((c,), jnp.int32) for _ in range(n_chunks)],
            [pltpu.VMEM((c, D), jnp.float32) for _ in range(n_chunks)],
            [pltpu.SemaphoreType.DMA for _ in range(n_chunks)],
            [pltpu.SemaphoreType.DMA for _ in range(n_chunks)],
        ],
    )
    def gather(table_hbm, idx_hbm, out_hbm, idx_vs, rows_vs, gsems, osems):
        wid = lax.axis_index("s") * nc + lax.axis_index("c")
        base = wid * b_per_w
        gathers = []
        for j in range(n_chunks):
            pltpu.sync_copy(idx_hbm.at[pl.ds(base + j * c, c)], idx_vs[j])
            gathers.append(
                pltpu.async_copy(table_hbm.at[idx_vs[j]], rows_vs[j], gsems[j]))
        outs = []
        for j in range(n_chunks):
            gathers[j].wait()
            outs.append(
                pltpu.async_copy(rows_vs[j], out_hbm.at[pl.ds(base + j * c, c)],
                                 osems[j]))
        for cp in outs:
            cp.wait()

    return gather


def _sc_gather(table, idx, b_total):
    return _make_sc_gather(table.shape[0], b_total)(table, idx)


def _shared_body(x_ref, ws1_ref, ws2_ref, out_ref):
    h = jnp.dot(x_ref[...], ws1_ref[...], preferred_element_type=jnp.float32)
    h = h / (1.0 + jnp.exp(-h))
    out_ref[...] = jnp.dot(h, ws2_ref[...], preferred_element_type=jnp.float32)


def _shared_expert_half(x_flat, Ws1h, Ws2h):
    BT = 512
    nt = T // BT
    hh = Ws1h.shape[1]
    return pl.pallas_call(
        _shared_body,
        grid=(nt,),
        in_specs=[
            pl.BlockSpec((BT, D), lambda i: (i, 0)),
            pl.BlockSpec((D, hh), lambda i: (0, 0)),
            pl.BlockSpec((hh, D), lambda i: (0, 0)),
        ],
        out_specs=pl.BlockSpec((BT, D), lambda i: (i, 0)),
        out_shape=jax.ShapeDtypeStruct((T, D), jnp.float32),
    )(x_flat, Ws1h, Ws2h)


def _shared_combine_body(x_ref, ws1_ref, ws2_ref, z1_ref, z2_ref, g_ref, out_ref):
    h = jnp.dot(x_ref[...], ws1_ref[...], preferred_element_type=jnp.float32)
    h = h / (1.0 + jnp.exp(-h))
    sh = jnp.dot(h, ws2_ref[...], preferred_element_type=jnp.float32)
    out_ref[...] = (sh + g_ref[:, 0:1] * z1_ref[...] + g_ref[:, 1:2] * z2_ref[...])


def _shared_combine(x_flat, Ws1, Ws2, z, g):
    BT = 256
    nt = T // BT
    return pl.pallas_call(
        _shared_combine_body,
        grid=(nt,),
        in_specs=[
            pl.BlockSpec((BT, D), lambda i: (i, 0)),
            pl.BlockSpec((D, H), lambda i: (0, 0)),
            pl.BlockSpec((H, D), lambda i: (0, 0)),
            pl.BlockSpec((BT, D), lambda i: (i, 0)),
            pl.BlockSpec((BT, D), lambda i: (i + nt, 0)),
            pl.BlockSpec((BT, K), lambda i: (i, 0)),
        ],
        out_specs=pl.BlockSpec((BT, D), lambda i: (i, 0)),
        out_shape=jax.ShapeDtypeStruct((T, D), jnp.float32),
    )(x_flat, Ws1, Ws2, z, z, g)


def kernel(x, Wg, Ws1, Ws2, We1, We2):
    x_flat = x.reshape(T, D)
    bf = jnp.bfloat16
    HH = H // 2
    g, pos, src, be = _gating_meta(x_flat, Wg)
    x_sorted = _sc_gather(x_flat, src, P)
    y_sorted = _grouped_matmul(x_sorted, We1, We2, be)
    z = _sc_gather(y_sorted, pos, NPAIR)
    out = _shared_combine(x_flat, Ws1, Ws2, z, g)
    return out.reshape(B, S, D)
